# baseline (speedup 1.0000x reference)
import functools

import jax
import jax.numpy as jnp
from jax import lax
from jax.experimental import pallas as pl
from jax.experimental.pallas import tpu as pltpu

WS = 32
NN = 10


def _sort_sublanes(x, n):
    i_idx = lax.broadcasted_iota(jnp.int32, (n, 1), 0)
    stage = 2
    while stage <= n:
        jd = stage // 2
        while jd >= 1:
            up = jnp.concatenate([x[n - jd:], x[:n - jd]], axis=0)
            dn = jnp.concatenate([x[jd:], x[:jd]], axis=0)
            bit = (i_idx & jd) != 0
            p = jnp.where(bit, up, dn)
            asc = (i_idx & stage) == 0
            takemin = bit != asc
            x = jnp.where(takemin, jnp.minimum(x, p), jnp.maximum(x, p))
            jd //= 2
        stage *= 2
    return x


def _ext_kernel(x_ref, o_ref):
    w = x_ref[0]
    n = float(WS)
    mu = jnp.sum(w, axis=0, keepdims=True) / n
    dev = w - mu
    ss = jnp.sum(dev * dev, axis=0, keepdims=True)
    astd = jnp.sqrt(ss / (n - 1.0))
    sd = jnp.sqrt(ss / n)
    z = dev / sd
    z2 = z * z
    m2 = jnp.sum(z2, axis=0, keepdims=True) / n
    m3 = jnp.sum(z2 * z, axis=0, keepdims=True) / n
    m4 = jnp.sum(z2 * z2, axis=0, keepdims=True) / n
    skew = m3 / (m2 * jnp.sqrt(m2))
    kurt = m4 / (m2 * m2) - 3.0
    s = _sort_sublanes(w, WS)
    amax = s[WS - 1:WS, :]
    amin = s[0:1, :]
    med = s[(WS - 1) // 2:(WS - 1) // 2 + 1, :]
    sdev = _sort_sublanes(jnp.abs(w - med), WS)
    meddev = sdev[(WS - 1) // 2:(WS - 1) // 2 + 1, :]
    o_ref[0] = jnp.concatenate(
        [amax, amin, mu, astd, skew, kurt, meddev], axis=0)


def _relieff_kernel(xr_ref, xc_ref, o_ref, d2s_ref, cnt_ref,
                    *, G, F, NF, scale):
    i = pl.program_id(0)
    xr = xr_ref[...]
    xc = xc_ref[...]
    d2 = jnp.zeros((G, F, F), jnp.float32)
    for d in range(NF):
        diff = xr[:, :, d:d + 1] - xc[:, d:d + 1, :]
        d2 = d2 + diff * diff
    d2s_ref[...] = d2.reshape(G * F, F)
    cnt_ref[...] = jnp.zeros((G * F, F), jnp.int32)
    KG = 8
    for k0 in range(0, F, KG):
        d2v = d2s_ref[...]
        inc = jnp.zeros((G * F, F), jnp.int32)
        for k in range(k0, min(k0 + KG, F)):
            inc = inc + (d2s_ref[:, k:k + 1] < d2v).astype(jnp.int32)
        cnt_ref[...] += inc
    cnt = cnt_ref[...]
    wgt = (jnp.where(cnt < NN, -1.0, jnp.where(cnt < 2 * NN, 1.0, 0.0))
           .reshape(G, F, F))
    lane8 = lax.broadcasted_iota(jnp.int32, (1, 1, 128), 2)
    acc = jnp.zeros((1, 1, 128), jnp.float32)
    for d in range(NF):
        da = jnp.abs(xr[:, :, d:d + 1] - xc[:, d:d + 1, :])
        s = jnp.sum(wgt * da)
        acc = acc + jnp.where(lane8 == d, s, 0.0)

    @pl.when(i == 0)
    def _init():
        o_ref[...] = jnp.zeros_like(o_ref)

    o_ref[...] += acc

    @pl.when(i == pl.num_programs(0) - 1)
    def _final():
        o_ref[...] *= scale


def _proj_kernel(xf_ref, wih_ref, whh_ref, bih_ref, bhh_ref, fcw_ref,
                 fcbt_ref, convw_ref, convb_ref, imp_ref,
                 o_ref, weff_ref, gi_ref, hs_ref, *, Tn, B, H, D, NF):
    C = 3
    for i in range(C):
        s = convw_ref[0, i] * wih_ref[:, 0:D]
        for o in range(1, C):
            s = s + convw_ref[o, i] * wih_ref[:, o * D:(o + 1) * D]
        weff_ref[:, i * D:(i + 1) * D] = s
    ones = jnp.ones((1, D), jnp.float32)
    brow = bih_ref[...]
    for o in range(C):
        srow = lax.dot_general(
            ones, wih_ref[:, o * D:(o + 1) * D],
            (((1,), (1,)), ((), ())), preferred_element_type=jnp.float32)
        brow = brow + convb_ref[0, o] * srow
    gi_ref[...] = lax.dot_general(
        xf_ref[...], weff_ref[...],
        (((1,), (1,)), ((), ())), preferred_element_type=jnp.float32) + brow

    bhh_row = bhh_ref[...]
    whh = whh_ref[...]

    def step(t, h):
        git = gi_ref[pl.ds(t * B, B), :]
        gh = lax.dot_general(
            h, whh, (((1,), (1,)), ((), ())),
            preferred_element_type=jnp.float32) + bhh_row
        i_r, i_z, i_n = git[:, :H], git[:, H:2 * H], git[:, 2 * H:]
        h_r, h_z, h_n = gh[:, :H], gh[:, H:2 * H], gh[:, 2 * H:]
        r = jax.nn.sigmoid(i_r + h_r)
        z = jax.nn.sigmoid(i_z + h_z)
        nn_ = jnp.tanh(i_n + r * h_n)
        h_new = (1.0 - z) * nn_ + z * h
        hs_ref[pl.ds(t * B, B), :] = h_new
        return h_new

    lax.fori_loop(0, Tn, step, jnp.zeros((B, H), jnp.float32))

    efft = imp_ref[0, 0] * fcw_ref[:, 0, :]
    for d in range(1, NF):
        efft = efft + imp_ref[0, d] * fcw_ref[:, d, :]
    bout = imp_ref[0, 0] * fcbt_ref[0:1, :]
    for d in range(1, NF):
        bout = bout + imp_ref[0, d] * fcbt_ref[d:d + 1, :]
    o_ref[...] = lax.dot_general(
        hs_ref[...], efft, (((1,), (1,)), ((), ())),
        preferred_element_type=jnp.float32) + bout


def kernel(x, y, conv_w, conv_b, w_ih, w_hh, b_ih, b_hh, fc_w, fc_b):
    B, C, T, F = x.shape
    Tn = T // WS
    NF = 7
    H = w_hh.shape[1]
    G3 = w_ih.shape[0]
    D = F * NF

    xw = (x.reshape(B, C, Tn, WS, F)
            .transpose(0, 1, 3, 2, 4)
            .reshape(B * C, WS, Tn * F))
    ext7 = pl.pallas_call(
        _ext_kernel,
        grid=(B * C,),
        in_specs=[pl.BlockSpec((1, WS, Tn * F), lambda i: (i, 0, 0))],
        out_specs=pl.BlockSpec((1, NF, Tn * F), lambda i: (i, 0, 0)),
        out_shape=jax.ShapeDtypeStruct((B * C, NF, Tn * F), jnp.float32),
    )(xw)
    ext5 = ext7.reshape(B, C, NF, Tn, F).transpose(0, 1, 3, 4, 2)

    nb = B * C * Tn
    xr = ext5.reshape(nb, F, NF)
    xc = xr.transpose(0, 2, 1)
    G = 2
    for cand in (64, 32, 16, 8, 4, 2):
        if nb % cand == 0:
            G = cand
            break
    scale = 1.0 / (NN * F * Tn * C)
    imp_raw = pl.pallas_call(
        functools.partial(_relieff_kernel, G=G, F=F, NF=NF, scale=scale),
        grid=(nb // G,),
        in_specs=[
            pl.BlockSpec((G, F, NF), lambda i: (i, 0, 0)),
            pl.BlockSpec((G, NF, F), lambda i: (i, 0, 0)),
        ],
        out_specs=pl.BlockSpec((1, 1, 128), lambda i: (0, 0, 0)),
        out_shape=jax.ShapeDtypeStruct((1, 1, 128), jnp.float32),
        scratch_shapes=[
            pltpu.VMEM((G * F, F), jnp.float32),
            pltpu.VMEM((G * F, F), jnp.int32),
        ],
    )(xr, xc)
    imp = imp_raw[0, :, :NF]

    xf_bh = ext5.reshape(B * Tn, C * F * NF)
    xf = (xf_bh.reshape(B, Tn, C * F * NF)
               .transpose(1, 0, 2)
               .reshape(Tn * B, C * F * NF))
    out = pl.pallas_call(
        functools.partial(_proj_kernel, Tn=Tn, B=B, H=H, D=D, NF=NF),
        in_specs=[
            pl.BlockSpec(memory_space=pltpu.VMEM),
            pl.BlockSpec(memory_space=pltpu.VMEM),
            pl.BlockSpec(memory_space=pltpu.VMEM),
            pl.BlockSpec(memory_space=pltpu.VMEM),
            pl.BlockSpec(memory_space=pltpu.VMEM),
            pl.BlockSpec(memory_space=pltpu.VMEM),
            pl.BlockSpec(memory_space=pltpu.VMEM),
            pl.BlockSpec(memory_space=pltpu.SMEM),
            pl.BlockSpec(memory_space=pltpu.SMEM),
            pl.BlockSpec(memory_space=pltpu.SMEM),
        ],
        out_specs=pl.BlockSpec(memory_space=pltpu.VMEM),
        out_shape=jax.ShapeDtypeStruct((Tn * B, C * F), jnp.float32),
        scratch_shapes=[
            pltpu.VMEM((G3, C * F * NF), jnp.float32),
            pltpu.VMEM((Tn * B, G3), jnp.float32),
            pltpu.VMEM((Tn * B, H), jnp.float32),
        ],
    )(xf, w_ih, w_hh, b_ih.reshape(1, G3), b_hh.reshape(1, G3),
      fc_w.reshape(C * F, NF, H), fc_b.reshape(C * F, NF).T,
      conv_w, conv_b.reshape(1, C), imp)
    return (out.reshape(Tn, B, C * F)
               .transpose(1, 0, 2)
               .reshape(B, Tn, C, F))


# speedup vs baseline: 1.1224x; 1.1224x over previous
import functools

import jax
import jax.numpy as jnp
from jax import lax
from jax.experimental import pallas as pl
from jax.experimental.pallas import tpu as pltpu

WS = 32
NN = 10


def _sort_sublanes(x, n):
    i_idx = lax.broadcasted_iota(jnp.int32, (n, 1), 0)
    stage = 2
    while stage <= n:
        jd = stage // 2
        while jd >= 1:
            up = jnp.concatenate([x[n - jd:], x[:n - jd]], axis=0)
            dn = jnp.concatenate([x[jd:], x[:jd]], axis=0)
            bit = (i_idx & jd) != 0
            p = jnp.where(bit, up, dn)
            asc = (i_idx & stage) == 0
            takemin = bit != asc
            x = jnp.where(takemin, jnp.minimum(x, p), jnp.maximum(x, p))
            jd //= 2
        stage *= 2
    return x


def _ext_kernel(x_ref, o_ref):
    w = x_ref[0]
    n = float(WS)
    mu = jnp.sum(w, axis=0, keepdims=True) / n
    dev = w - mu
    ss = jnp.sum(dev * dev, axis=0, keepdims=True)
    astd = jnp.sqrt(ss / (n - 1.0))
    sd = jnp.sqrt(ss / n)
    z = dev / sd
    z2 = z * z
    m2 = jnp.sum(z2, axis=0, keepdims=True) / n
    m3 = jnp.sum(z2 * z, axis=0, keepdims=True) / n
    m4 = jnp.sum(z2 * z2, axis=0, keepdims=True) / n
    skew = m3 / (m2 * jnp.sqrt(m2))
    kurt = m4 / (m2 * m2) - 3.0
    s = _sort_sublanes(w, WS)
    amax = s[WS - 1:WS, :]
    amin = s[0:1, :]
    med = s[(WS - 1) // 2:(WS - 1) // 2 + 1, :]
    sdev = _sort_sublanes(jnp.abs(w - med), WS)
    meddev = sdev[(WS - 1) // 2:(WS - 1) // 2 + 1, :]
    zrow = jnp.zeros_like(mu)
    st8 = jnp.concatenate(
        [amax, amin, mu, astd, skew, kurt, meddev, zrow], axis=0)
    o_ref[0] = jnp.transpose(st8)[:, :7]


def _relieff_kernel(xr_ref, xc_ref, o_ref, *, G, F, NF, scale):
    i = pl.program_id(0)
    xr = xr_ref[...]
    xc = xc_ref[...]
    d2 = jnp.zeros((G, F, F), jnp.float32)
    for d in range(NF):
        diff = xr[:, :, d:d + 1] - xc[:, d:d + 1, :]
        d2 = d2 + diff * diff
    dist = jnp.sqrt(jnp.maximum(d2, 0.0)).reshape(G * F, F)
    jcol = lax.broadcasted_iota(jnp.int32, (G * F, F), 1)
    cnt = jnp.zeros((G * F, F), jnp.int32)
    for k in range(F):
        dk = dist[:, k:k + 1]
        lt = (dk < dist).astype(jnp.int32)
        tie = jnp.logical_and(dk == dist, k < jcol).astype(jnp.int32)
        cnt = cnt + lt + tie
    wgt = (jnp.where(cnt < NN, -1.0, jnp.where(cnt < 2 * NN, 1.0, 0.0))
           .reshape(G, F, F))
    lane8 = lax.broadcasted_iota(jnp.int32, (1, 1, 128), 2)
    acc = jnp.zeros((1, 1, 128), jnp.float32)
    for d in range(NF):
        da = jnp.abs(xr[:, :, d:d + 1] - xc[:, d:d + 1, :])
        s = jnp.sum(wgt * da)
        acc = acc + jnp.where(lane8 == d, s, 0.0)

    @pl.when(i == 0)
    def _init():
        o_ref[...] = jnp.zeros_like(o_ref)

    o_ref[...] += acc

    @pl.when(i == pl.num_programs(0) - 1)
    def _final():
        o_ref[...] *= scale


def _proj_kernel(xf_ref, wih_ref, whh_ref, bih_ref, bhh_ref, fcw_ref,
                 fcbt_ref, convw_ref, convb_ref, imp_ref,
                 o_ref, weff_ref, gi_ref, hs_ref, *, Tn, B, H, D, NF):
    C = 3
    for i in range(C):
        s = convw_ref[0, i] * wih_ref[:, 0:D]
        for o in range(1, C):
            s = s + convw_ref[o, i] * wih_ref[:, o * D:(o + 1) * D]
        weff_ref[:, i * D:(i + 1) * D] = s
    ones = jnp.ones((1, D), jnp.float32)
    brow = bih_ref[...]
    for o in range(C):
        srow = lax.dot_general(
            ones, wih_ref[:, o * D:(o + 1) * D],
            (((1,), (1,)), ((), ())), preferred_element_type=jnp.float32)
        brow = brow + convb_ref[0, o] * srow
    gi_ref[...] = lax.dot_general(
        xf_ref[...], weff_ref[...],
        (((1,), (1,)), ((), ())), preferred_element_type=jnp.float32) + brow

    bhh_row = bhh_ref[...]
    whh = whh_ref[...]

    def step(t, h):
        git = gi_ref[pl.ds(t * B, B), :]
        gh = lax.dot_general(
            h, whh, (((1,), (1,)), ((), ())),
            preferred_element_type=jnp.float32) + bhh_row
        i_r, i_z, i_n = git[:, :H], git[:, H:2 * H], git[:, 2 * H:]
        h_r, h_z, h_n = gh[:, :H], gh[:, H:2 * H], gh[:, 2 * H:]
        r = jax.nn.sigmoid(i_r + h_r)
        z = jax.nn.sigmoid(i_z + h_z)
        nn_ = jnp.tanh(i_n + r * h_n)
        h_new = (1.0 - z) * nn_ + z * h
        hs_ref[pl.ds(t * B, B), :] = h_new
        return h_new

    lax.fori_loop(0, Tn, step, jnp.zeros((B, H), jnp.float32))

    efft = imp_ref[0, 0] * fcw_ref[:, 0, :]
    for d in range(1, NF):
        efft = efft + imp_ref[0, d] * fcw_ref[:, d, :]
    bout = imp_ref[0, 0] * fcbt_ref[0:1, :]
    for d in range(1, NF):
        bout = bout + imp_ref[0, d] * fcbt_ref[d:d + 1, :]
    o_ref[...] = lax.dot_general(
        hs_ref[...], efft, (((1,), (1,)), ((), ())),
        preferred_element_type=jnp.float32) + bout


def kernel(x, y, conv_w, conv_b, w_ih, w_hh, b_ih, b_hh, fc_w, fc_b):
    B, C, T, F = x.shape
    Tn = T // WS
    NF = 7
    H = w_hh.shape[1]
    G3 = w_ih.shape[0]
    D = F * NF

    xw = (x.reshape(B, C, Tn, WS, F)
            .transpose(0, 1, 3, 2, 4)
            .reshape(B * C, WS, Tn * F))
    ext_t = pl.pallas_call(
        _ext_kernel,
        grid=(B * C,),
        in_specs=[pl.BlockSpec((1, WS, Tn * F), lambda i: (i, 0, 0))],
        out_specs=pl.BlockSpec((1, Tn * F, NF), lambda i: (i, 0, 0)),
        out_shape=jax.ShapeDtypeStruct((B * C, Tn * F, NF), jnp.float32),
    )(xw)

    nb = B * C * Tn
    xr = ext_t.reshape(nb, F, NF)
    xc = xr.transpose(0, 2, 1)
    G = 2
    for cand in (64, 32, 16, 8, 4, 2):
        if nb % cand == 0:
            G = cand
            break
    scale = 1.0 / (NN * F * Tn * C)
    imp_raw = pl.pallas_call(
        functools.partial(_relieff_kernel, G=G, F=F, NF=NF, scale=scale),
        grid=(nb // G,),
        in_specs=[
            pl.BlockSpec((G, F, NF), lambda i: (i, 0, 0)),
            pl.BlockSpec((G, NF, F), lambda i: (i, 0, 0)),
        ],
        out_specs=pl.BlockSpec((1, 1, 128), lambda i: (0, 0, 0)),
        out_shape=jax.ShapeDtypeStruct((1, 1, 128), jnp.float32),
    )(xr, xc)
    imp = imp_raw[0, :, :NF]

    xf = (ext_t.reshape(B, Tn, C * F * NF)
               .transpose(1, 0, 2)
               .reshape(Tn * B, C * F * NF))
    out = pl.pallas_call(
        functools.partial(_proj_kernel, Tn=Tn, B=B, H=H, D=D, NF=NF),
        in_specs=[
            pl.BlockSpec(memory_space=pltpu.VMEM),
            pl.BlockSpec(memory_space=pltpu.VMEM),
            pl.BlockSpec(memory_space=pltpu.VMEM),
            pl.BlockSpec(memory_space=pltpu.VMEM),
            pl.BlockSpec(memory_space=pltpu.VMEM),
            pl.BlockSpec(memory_space=pltpu.VMEM),
            pl.BlockSpec(memory_space=pltpu.VMEM),
            pl.BlockSpec(memory_space=pltpu.SMEM),
            pl.BlockSpec(memory_space=pltpu.SMEM),
            pl.BlockSpec(memory_space=pltpu.SMEM),
        ],
        out_specs=pl.BlockSpec(memory_space=pltpu.VMEM),
        out_shape=jax.ShapeDtypeStruct((Tn * B, C * F), jnp.float32),
        scratch_shapes=[
            pltpu.VMEM((G3, C * F * NF), jnp.float32),
            pltpu.VMEM((Tn * B, G3), jnp.float32),
            pltpu.VMEM((Tn * B, H), jnp.float32),
        ],
    )(xf, w_ih, w_hh, b_ih.reshape(1, G3), b_hh.reshape(1, G3),
      fc_w.reshape(C * F, NF, H), fc_b.reshape(C * F, NF).T,
      conv_w, conv_b.reshape(1, C), imp)
    return (out.reshape(Tn, B, C * F)
               .transpose(1, 0, 2)
               .reshape(B, Tn, C, F))
